# Initial kernel scaffold; baseline (speedup 1.0000x reference)
import jax
import jax.numpy as jnp
from jax.experimental import pallas as pl


def kernel(weights, spacing_bins, nears, fars):
    raise NotImplementedError("write your pallas kernel here")


# speedup vs baseline: 13.6185x; 13.6185x over previous
import functools

import jax
import jax.numpy as jnp
from jax import lax
from jax.experimental import pallas as pl
from jax.experimental.pallas import tpu as pltpu
from jax.experimental.pallas import tpu_sc as plsc

_HIST_PAD = 0.01
_EPS = 1e-5
_S = 128
_NBINS = 129
_OUTW = 258
_L = 16


def _sc_sampler(gpw, w_hbm, b_hbm, n_hbm, f_hbm, out_hbm,
                wbuf, binsbuf, nbuf, fbuf, cdfraw, cdfbuf, invdbuf, flagbuf,
                outbuf):
    nc = 2
    wid = lax.axis_index("s") * nc + lax.axis_index("c")
    lane = lax.iota(jnp.int32, _L)
    base_w = lane * _S
    base_c = lane * _NBINS
    base_o = lane * _OUTW
    zero_f = jnp.zeros((_L,), jnp.float32)
    zero_i = jnp.zeros((_L,), jnp.int32)

    @plsc.parallel_loop(0, _L * _OUTW, step=_L, unroll=4)
    def _(o):
        flagbuf[pl.ds(o, _L)] = zero_i

    def group_body(t, _):
        g = wid * gpw + t
        pltpu.sync_copy(w_hbm.at[pl.ds(g * (_L * _S), _L * _S)], wbuf)
        pltpu.sync_copy(b_hbm.at[pl.ds(g * (_L * _NBINS), _L * _NBINS)], binsbuf)
        pltpu.sync_copy(n_hbm.at[pl.ds(g * _L, _L)], nbuf)
        pltpu.sync_copy(f_hbm.at[pl.ds(g * _L, _L)], fbuf)

        def _cumsum_body(i, run):
            wv = plsc.load_gather(wbuf, [base_w + i]) + _HIST_PAD
            run = run + wv
            plsc.store_scatter(cdfraw, [base_w + i], run)
            return run

        wsum = plsc.parallel_loop(0, _S, unroll=4, carry=zero_f)(_cumsum_body)

        padding = jnp.maximum(_EPS - wsum, 0.0)
        inv = 1.0 / (wsum + padding)
        padper = padding * (1.0 / _S)

        plsc.store_scatter(cdfbuf, [base_c], zero_f)
        plsc.store_scatter(invdbuf, [base_c + _S], zero_f)
        plsc.store_scatter(flagbuf, [base_o], jnp.ones((_L,), jnp.int32))

        @plsc.parallel_loop(1, _NBINS, unroll=4)
        def _(i):
            craw = plsc.load_gather(cdfraw, [base_w + (i - 1)])
            i_f = i.astype(jnp.float32)
            c = jnp.minimum(1.0, (craw + i_f * padper) * inv)
            craw_p = plsc.load_gather(cdfraw, [base_w + jnp.maximum(i - 2, 0)])
            cp = jnp.where(
                i == 1, 0.0,
                jnp.minimum(1.0, (craw_p + (i_f - 1.0) * padper) * inv))
            v = jnp.maximum(c * float(_NBINS) - 0.5, 0.0)
            ti = v.astype(jnp.int32)
            cnt = ti + (ti.astype(jnp.float32) < v).astype(jnp.int32)
            cnt = jnp.minimum(cnt, _NBINS)
            denom = c - cp
            invd = jnp.where(denom > 0.0, 1.0 / denom, 0.0)
            plsc.store_scatter(cdfbuf, [base_c + i], c)
            plsc.store_scatter(invdbuf, [(base_c + i) - 1], invd)
            plsc.store_scatter(flagbuf, [(base_o + cnt) + i],
                               jnp.ones((_L,), jnp.int32))

        near = nbuf[...]
        fmn = fbuf[...] - near
        b1_0 = plsc.load_gather(binsbuf, [base_c])
        init = (jnp.full((_L,), -1, jnp.int32), b1_0, b1_0, zero_f, zero_f)

        def _walk_body(k, carry):
            seg, b0, b1, c0, c1 = carry
            isb = plsc.load_gather(flagbuf, [base_o + k])
            plsc.store_scatter(flagbuf, [base_o + k], zero_i)
            seg = seg + isb
            mb = isb > 0
            b0 = jnp.where(mb, b1, b0)
            c0 = jnp.where(mb, c1, c0)
            segn = jnp.minimum(seg + 1, _S)
            b1 = plsc.load_gather(binsbuf, [base_c + segn])
            c1 = plsc.load_gather(cdfbuf, [base_c + segn])
            invd = plsc.load_gather(invdbuf, [base_c + seg])
            u = ((k - seg).astype(jnp.float32) - 0.5) * (1.0 / _NBINS)
            t = jnp.maximum((u - c0) * invd, 0.0)
            val = b0 + t * (b1 - b0)
            eu = val * fmn + near
            plsc.store_scatter(outbuf, [base_o + k], eu)
            return (seg, b0, b1, c0, c1)

        plsc.parallel_loop(0, _OUTW, unroll=4, carry=init)(_walk_body)
        pltpu.sync_copy(outbuf, out_hbm.at[pl.ds(g * (_L * _OUTW), _L * _OUTW)])
        return 0

    lax.fori_loop(0, gpw, group_body, 0)


def kernel(weights, spacing_bins, nears, fars):
    r = weights.shape[0]
    info = plsc.get_sparse_core_info()
    nw = info.num_cores * info.num_subcores
    gpw = (r // _L) // nw
    mesh = plsc.VectorSubcoreMesh(core_axis_name="c", subcore_axis_name="s")
    run = pl.kernel(
        functools.partial(_sc_sampler, gpw),
        out_type=jax.ShapeDtypeStruct((r * _OUTW,), jnp.float32),
        mesh=mesh,
        scratch_types=[
            pltpu.VMEM((_L * _S,), jnp.float32),
            pltpu.VMEM((_L * _NBINS,), jnp.float32),
            pltpu.VMEM((_L,), jnp.float32),
            pltpu.VMEM((_L,), jnp.float32),
            pltpu.VMEM((_L * _S,), jnp.float32),
            pltpu.VMEM((_L * _NBINS,), jnp.float32),
            pltpu.VMEM((_L * _NBINS,), jnp.float32),
            pltpu.VMEM((_L * _OUTW,), jnp.int32),
            pltpu.VMEM((_L * _OUTW,), jnp.float32),
        ],
        compiler_params=pltpu.CompilerParams(needs_layout_passes=False),
        name="error_bounded_sampler_sc",
    )
    out = run(weights.reshape(r * _S),
              spacing_bins.reshape(r * _NBINS),
              nears.reshape(r),
              fars.reshape(r))
    return out.reshape(r, _OUTW)
